# baseline (speedup 1.0000x reference)
import functools

import jax
import jax.numpy as jnp
import numpy as np
from jax import lax
from jax.experimental import pallas as pl
from jax.experimental.pallas import tpu as pltpu
from jax.experimental.pallas import tpu_sc as plsc

B = 8
N = 4096
NPOINT = 512
NSAMPLE = 32
R2 = np.float32(0.2 * 0.2)
EPS = np.float32(1e-5)
NROWS = B * NPOINT * NSAMPLE
NQ = B * NPOINT
BIG = np.int32(1 << 30)


def _fps_body(xyz_ref, nxt_ref, idx_ref):
    X = xyz_ref[0]
    Y = xyz_ref[1]
    Z = xyz_ref[2]
    lanes = lax.broadcasted_iota(jnp.int32, (B, N), 1)
    slots = lax.broadcasted_iota(jnp.int32, (B, NPOINT), 1)

    def step(i, carry):
        dist, far, cxs, cys, czs, idxs = carry
        oh = lanes == far
        cx = jnp.sum(jnp.where(oh, X, 0.0), axis=1, keepdims=True)
        cy = jnp.sum(jnp.where(oh, Y, 0.0), axis=1, keepdims=True)
        cz = jnp.sum(jnp.where(oh, Z, 0.0), axis=1, keepdims=True)
        hit = slots == i
        zf = jnp.zeros((B, NPOINT), jnp.float32)
        cxs = jnp.where(hit, cx + zf, cxs)
        cys = jnp.where(hit, cy + zf, cys)
        czs = jnp.where(hit, cz + zf, czs)
        idxs = jnp.where(hit, far + jnp.zeros((B, NPOINT), jnp.int32), idxs)
        dx = X - cx
        dy = Y - cy
        dz = Z - cz
        d = (dx * dx + dy * dy) + dz * dz
        dist = jnp.minimum(dist, d)
        m = jnp.max(dist, axis=1, keepdims=True)
        far = jnp.min(jnp.where(dist == m, lanes, BIG), axis=1, keepdims=True)
        return dist, far, cxs, cys, czs, idxs

    d0 = jnp.full((B, N), 1e10, dtype=jnp.float32)
    f0 = jnp.zeros((B, 1), dtype=jnp.int32)
    z0 = jnp.zeros((B, NPOINT), dtype=jnp.float32)
    i0 = jnp.zeros((B, NPOINT), dtype=jnp.int32)
    carry1 = step(0, (d0, f0, z0, z0, z0, i0))
    _, _, cxs, cys, czs, idxs = lax.fori_loop(1, NPOINT, step, carry1)
    nxt_ref[:, 0, :] = cxs
    nxt_ref[:, 1, :] = cys
    nxt_ref[:, 2, :] = czs
    idx_ref[...] = idxs


def _fps(xyz_t):
    return pl.pallas_call(
        _fps_body,
        out_shape=(
            jax.ShapeDtypeStruct((B, 3, NPOINT), jnp.float32),
            jax.ShapeDtypeStruct((B, NPOINT), jnp.int32),
        ),
    )(xyz_t)


def _proj_body(a_ref, w_ref, o_ref):
    o_ref[...] = lax.dot_general(
        a_ref[...], w_ref[...], (((1,), (1,)), ((), ())),
        preferred_element_type=jnp.float32)


def _proj(a_pad, w_pad):
    rtot = a_pad.shape[0]
    blk = 1024
    return pl.pallas_call(
        _proj_body,
        grid=(rtot // blk,),
        in_specs=[
            pl.BlockSpec((blk, 128), lambda i: (i, 0)),
            pl.BlockSpec((128, 128), lambda i: (0, 0)),
        ],
        out_specs=pl.BlockSpec((blk, 128), lambda i: (i, 0)),
        out_shape=jax.ShapeDtypeStruct((rtot, 128), jnp.float32),
    )(a_pad, w_pad)


QB = 256
IMAX = np.int32(0x7FFFFFFF)
KFIX = np.float32((1 << 19) / (0.2 * 0.2))


def _bq_body(xyz_ref, q_ref, sel_ref):
    b = pl.program_id(0)
    X = xyz_ref[0, 0:1, :]
    Y = xyz_ref[0, 1:2, :]
    Z = xyz_ref[0, 2:3, :]
    qx = q_ref[0, :, 0:1]
    qy = q_ref[0, :, 1:2]
    qz = q_ref[0, :, 2:3]
    dx = qx - X
    dy = qy - Y
    dz = qz - Z
    d = (dx * dx + dy * dy) + dz * dz
    lanes = lax.broadcasted_iota(jnp.int32, (QB, N), 1)
    db = lax.bitcast_convert_type(d, jnp.int32)
    keyfull = jnp.bitwise_or(jnp.bitwise_and(db, np.int32(~0xFFF)), lanes)
    closest = jnp.bitwise_and(
        jnp.min(keyfull, axis=1, keepdims=True), np.int32(0xFFF))
    within = d < R2
    v = jnp.sum(within.astype(jnp.int32), axis=1, keepdims=True)
    df = jnp.minimum((d * KFIX).astype(jnp.int32), np.int32((1 << 19) - 1))
    keyfix = jnp.bitwise_or(lax.shift_left(df, 12), lanes)
    keys = jnp.where(within, keyfix, IMAX)
    cols = lax.broadcasted_iota(jnp.int32, (QB, NSAMPLE), 1)

    def pick_step(k, carry):
        keys, sel = carry
        mk = jnp.min(keys, axis=1, keepdims=True)
        p = jnp.bitwise_and(mk, np.int32(0xFFF))
        sel = jnp.where(cols == k, p + jnp.zeros((QB, NSAMPLE), jnp.int32), sel)
        keys = jnp.where(keys == mk, IMAX, keys)
        return keys, sel

    sel0 = jnp.zeros((QB, NSAMPLE), dtype=jnp.int32)
    carry1 = pick_step(0, (keys, sel0))
    _, sel = lax.fori_loop(1, NSAMPLE, pick_step, carry1)

    mod = cols % jnp.maximum(v, 1)
    zi = jnp.zeros((QB, NSAMPLE), dtype=jnp.int32)
    res = zi
    for j in range(NSAMPLE):
        res = jnp.where(mod == j, sel[:, j:j + 1] + zi, res)
    res = jnp.where(v == 0, closest + zi, res)
    sel_ref[0, 0] = res + b * N


def _ball_query(xyz_t, new_xyz):
    return pl.pallas_call(
        _bq_body,
        grid=(B, NPOINT // QB),
        in_specs=[
            pl.BlockSpec((1, 3, N), lambda b, q: (b, 0, 0)),
            pl.BlockSpec((1, QB, 3), lambda b, q: (b, q, 0)),
        ],
        out_specs=pl.BlockSpec((1, 1, QB, NSAMPLE), lambda b, q: (b, q, 0, 0)),
        out_shape=jax.ShapeDtypeStruct((B, NPOINT // QB, QB, NSAMPLE), jnp.int32),
    )(xyz_t, new_xyz)


GCHUNK = 128
GBUF = 4


def _sc_gather(p_flat, ids2d):
    info = plsc.get_sparse_core_info()
    nc, ns = info.num_cores, info.num_subcores
    nw = nc * ns
    chunks_per_w = NROWS // nw // GCHUNK
    outer = chunks_per_w // GBUF
    mesh = plsc.VectorSubcoreMesh(core_axis_name="c", subcore_axis_name="s")

    @functools.partial(
        pl.kernel,
        mesh=mesh,
        out_type=jax.ShapeDtypeStruct((NROWS, 128), jnp.float32),
        scratch_types=[
            pltpu.VMEM((chunks_per_w, GCHUNK), jnp.int32),
            pltpu.VMEM((GBUF, GCHUNK, 128), jnp.float32),
            pltpu.SemaphoreType.DMA,
            pltpu.SemaphoreType.DMA,
        ],
    )
    def k(p_hbm, idx_hbm, out_hbm, idx_v, rows_v, gsem, wsem):
        wid = lax.axis_index("s") * nc + lax.axis_index("c")
        pltpu.sync_copy(idx_hbm.at[pl.ds(wid * chunks_per_w, chunks_per_w)], idx_v)

        def body(g, _):
            copies = []
            for bb in range(GBUF):
                copies.append(pltpu.async_copy(
                    p_hbm.at[idx_v.at[g * GBUF + bb]], rows_v.at[bb], gsem))
            wcopies = []
            for bb in range(GBUF):
                copies[bb].wait()
                base = (wid * chunks_per_w + g * GBUF + bb) * GCHUNK
                wcopies.append(pltpu.async_copy(
                    rows_v.at[bb], out_hbm.at[pl.ds(base, GCHUNK)], wsem))
            for bb in range(GBUF):
                wcopies[bb].wait()
            return 0

        lax.fori_loop(0, outer, body, 0)

    return k(p_flat, ids2d)


MBLK = 64


def _stats0_body(g_ref, c_ref, s_ref):
    y = g_ref[..., :64] - c_ref[..., :64][:, None, :]
    s = jnp.sum(y, axis=(0, 1))
    sq = jnp.sum(y * y, axis=(0, 1))
    acc = jnp.stack([s, sq])

    @pl.when(pl.program_id(0) == 0)
    def _():
        s_ref[...] = jnp.zeros_like(s_ref)

    s_ref[...] += acc


def _stats0(g, cq):
    return pl.pallas_call(
        _stats0_body,
        grid=(NQ // MBLK,),
        in_specs=[
            pl.BlockSpec((MBLK, NSAMPLE, 128), lambda i: (i, 0, 0)),
            pl.BlockSpec((MBLK, 128), lambda i: (i, 0)),
        ],
        out_specs=pl.BlockSpec((2, 64), lambda i: (0, 0)),
        out_shape=jax.ShapeDtypeStruct((2, 64), jnp.float32),
    )(g, cq)


def _norm_consts(sums, gamma, beta):
    cnt = np.float32(NROWS)
    mean = sums[0] / cnt
    var = sums[1] / cnt - mean * mean
    scale = gamma / jnp.sqrt(var + EPS)
    shift = beta - mean * scale
    return scale, shift


def _layer_body(sub_cq, cin, g_ref, c_ref, s_ref, w_ref, gm_ref, bt_ref, o_ref, acc_ref):
    scale, shift = _norm_consts(s_ref[...], gm_ref[0], bt_ref[0])
    x = g_ref[..., :cin]
    if sub_cq:
        x = x - c_ref[..., :cin][:, None, :]
    h = jnp.maximum(x * scale + shift, 0.0)
    h2 = h.reshape(MBLK * NSAMPLE, cin)
    y = lax.dot_general(h2, w_ref[...], (((1,), (1,)), ((), ())),
                        preferred_element_type=jnp.float32)
    cout = y.shape[-1]
    o_ref[...] = y.reshape(MBLK, NSAMPLE, cout)
    s = jnp.sum(y, axis=0)
    sq = jnp.sum(y * y, axis=0)
    acc = jnp.stack([s, sq])

    @pl.when(pl.program_id(0) == 0)
    def _():
        acc_ref[...] = jnp.zeros_like(acc_ref)

    acc_ref[...] += acc


def _layer(g, cq, sums, w, gamma, beta, sub_cq, cin):
    cout = w.shape[0]
    gw = g.shape[-1]
    cw = cq.shape[-1]
    in_specs = [
        pl.BlockSpec((MBLK, NSAMPLE, gw), lambda i: (i, 0, 0)),
        pl.BlockSpec((MBLK, cw), lambda i: (i, 0)),
        pl.BlockSpec((2, cin), lambda i: (0, 0)),
        pl.BlockSpec((cout, cin), lambda i: (0, 0)),
        pl.BlockSpec((1, cin), lambda i: (0, 0)),
        pl.BlockSpec((1, cin), lambda i: (0, 0)),
    ]
    return pl.pallas_call(
        functools.partial(_layer_body, sub_cq, cin),
        grid=(NQ // MBLK,),
        in_specs=in_specs,
        out_specs=(
            pl.BlockSpec((MBLK, NSAMPLE, cout), lambda i: (i, 0, 0)),
            pl.BlockSpec((2, cout), lambda i: (0, 0)),
        ),
        out_shape=(
            jax.ShapeDtypeStruct((NQ, NSAMPLE, cout), jnp.float32),
            jax.ShapeDtypeStruct((2, cout), jnp.float32),
        ),
    )(g, cq, sums, w, gamma.reshape(1, cin), beta.reshape(1, cin))


def _final_body(g_ref, s_ref, gm_ref, bt_ref, o_ref):
    scale, shift = _norm_consts(s_ref[...], gm_ref[0], bt_ref[0])
    h = jnp.maximum(g_ref[...] * scale + shift, 0.0)
    o_ref[...] = jnp.max(h, axis=1)


def _final(y2, sums, gamma, beta):
    c = y2.shape[-1]
    return pl.pallas_call(
        _final_body,
        grid=(NQ // MBLK,),
        in_specs=[
            pl.BlockSpec((MBLK, NSAMPLE, c), lambda i: (i, 0, 0)),
            pl.BlockSpec((2, c), lambda i: (0, 0)),
            pl.BlockSpec((1, c), lambda i: (0, 0)),
            pl.BlockSpec((1, c), lambda i: (0, 0)),
        ],
        out_specs=pl.BlockSpec((MBLK, c), lambda i: (i, 0)),
        out_shape=jax.ShapeDtypeStruct((NQ, c), jnp.float32),
    )(y2, sums, gamma.reshape(1, c), beta.reshape(1, c))


def kernel(xyz, points, W0, b0, g0, beta0, W1, b1, g1, beta1, W2, b2, g2, beta2):
    del b0, b1, b2
    xyz_t3 = jnp.transpose(xyz, (2, 0, 1))
    new_xyz_t, _ = _fps(xyz_t3)
    new_xyz = jnp.transpose(new_xyz_t, (0, 2, 1))

    a_pts = jnp.concatenate([xyz, points], axis=-1).reshape(B * N, 3 + points.shape[-1])
    a_pts = jnp.pad(a_pts, ((0, 0), (0, 128 - a_pts.shape[1])))
    a_ctr = jnp.pad(new_xyz.reshape(NQ, 3), ((0, 0), (0, 125)))
    a_all = jnp.concatenate([a_pts, a_ctr], axis=0)
    w0_pad = jnp.pad(W0, ((0, 128 - W0.shape[0]), (0, 128 - W0.shape[1])))
    p_all = _proj(a_all, w0_pad)
    p_flat = p_all[:B * N]
    cq = p_all[B * N:]

    xyz_t = jnp.transpose(xyz, (0, 2, 1))
    selg = _ball_query(xyz_t, new_xyz)
    g = _sc_gather(p_flat, selg.reshape(NROWS // GCHUNK, GCHUNK)).reshape(
        NQ, NSAMPLE, 128)

    sums0 = _stats0(g, cq)
    y1, sums1 = _layer(g, cq, sums0, W1, g0, beta0, True, 64)
    y2, sums2 = _layer(y1, cq, sums1, W2, g1, beta1, False, 64)
    out = _final(y2, sums2, g2, beta2)

    new_points = out.reshape(B, NPOINT, out.shape[-1])
    return new_xyz, new_points


# speedup vs baseline: 1.0132x; 1.0132x over previous
import functools

import jax
import jax.numpy as jnp
import numpy as np
from jax import lax
from jax.experimental import pallas as pl
from jax.experimental.pallas import tpu as pltpu
from jax.experimental.pallas import tpu_sc as plsc

B = 8
N = 4096
NPOINT = 512
NSAMPLE = 32
R2 = np.float32(0.2 * 0.2)
EPS = np.float32(1e-5)
NROWS = B * NPOINT * NSAMPLE
NQ = B * NPOINT
BIG = np.int32(1 << 30)


def _fps_body(xyz_ref, nxt_ref, idx_ref):
    X = xyz_ref[0]
    Y = xyz_ref[1]
    Z = xyz_ref[2]
    lanes = lax.broadcasted_iota(jnp.int32, (B, N), 1)
    slots = lax.broadcasted_iota(jnp.int32, (B, NPOINT), 1)

    def step(i, carry):
        dist, far, cxs, cys, czs, idxs = carry
        oh = lanes == far
        cx = jnp.sum(jnp.where(oh, X, 0.0), axis=1, keepdims=True)
        cy = jnp.sum(jnp.where(oh, Y, 0.0), axis=1, keepdims=True)
        cz = jnp.sum(jnp.where(oh, Z, 0.0), axis=1, keepdims=True)
        hit = slots == i
        zf = jnp.zeros((B, NPOINT), jnp.float32)
        cxs = jnp.where(hit, cx + zf, cxs)
        cys = jnp.where(hit, cy + zf, cys)
        czs = jnp.where(hit, cz + zf, czs)
        idxs = jnp.where(hit, far + jnp.zeros((B, NPOINT), jnp.int32), idxs)
        dx = X - cx
        dy = Y - cy
        dz = Z - cz
        d = (dx * dx + dy * dy) + dz * dz
        dist = jnp.minimum(dist, d)
        m = jnp.max(dist, axis=1, keepdims=True)
        far = jnp.min(jnp.where(dist == m, lanes, BIG), axis=1, keepdims=True)
        return dist, far, cxs, cys, czs, idxs

    d0 = jnp.full((B, N), 1e10, dtype=jnp.float32)
    f0 = jnp.zeros((B, 1), dtype=jnp.int32)
    z0 = jnp.zeros((B, NPOINT), dtype=jnp.float32)
    i0 = jnp.zeros((B, NPOINT), dtype=jnp.int32)
    carry1 = step(0, (d0, f0, z0, z0, z0, i0))
    _, _, cxs, cys, czs, idxs = lax.fori_loop(1, NPOINT, step, carry1)
    nxt_ref[:, 0, :] = cxs
    nxt_ref[:, 1, :] = cys
    nxt_ref[:, 2, :] = czs
    idx_ref[...] = idxs


def _fps(xyz_t):
    return pl.pallas_call(
        _fps_body,
        out_shape=(
            jax.ShapeDtypeStruct((B, 3, NPOINT), jnp.float32),
            jax.ShapeDtypeStruct((B, NPOINT), jnp.int32),
        ),
    )(xyz_t)


def _proj_body(a_ref, w_ref, o_ref):
    o_ref[...] = lax.dot_general(
        a_ref[...], w_ref[...], (((1,), (1,)), ((), ())),
        preferred_element_type=jnp.float32)


def _proj(a_pad, w_pad):
    rtot = a_pad.shape[0]
    blk = 1024
    return pl.pallas_call(
        _proj_body,
        grid=(rtot // blk,),
        in_specs=[
            pl.BlockSpec((blk, 128), lambda i: (i, 0)),
            pl.BlockSpec((128, 128), lambda i: (0, 0)),
        ],
        out_specs=pl.BlockSpec((blk, 128), lambda i: (i, 0)),
        out_shape=jax.ShapeDtypeStruct((rtot, 128), jnp.float32),
    )(a_pad, w_pad)


QB = 256
IMAX = np.int32(0x7FFFFFFF)
KFIX = np.float32((1 << 19) / (0.2 * 0.2))


def _bq_body(xyz_ref, q_ref, sel_ref):
    b = pl.program_id(0)
    X = xyz_ref[0, 0:1, :]
    Y = xyz_ref[0, 1:2, :]
    Z = xyz_ref[0, 2:3, :]
    qx = q_ref[0, :, 0:1]
    qy = q_ref[0, :, 1:2]
    qz = q_ref[0, :, 2:3]
    dx = qx - X
    dy = qy - Y
    dz = qz - Z
    d = (dx * dx + dy * dy) + dz * dz
    lanes = lax.broadcasted_iota(jnp.int32, (QB, N), 1)
    db = lax.bitcast_convert_type(d, jnp.int32)
    keyfull = jnp.bitwise_or(jnp.bitwise_and(db, np.int32(~0xFFF)), lanes)
    closest = jnp.bitwise_and(
        jnp.min(keyfull, axis=1, keepdims=True), np.int32(0xFFF))
    within = d < R2
    v = jnp.sum(within.astype(jnp.int32), axis=1, keepdims=True)
    df = jnp.minimum((d * KFIX).astype(jnp.int32), np.int32((1 << 19) - 1))
    keyfix = jnp.bitwise_or(lax.shift_left(df, 12), lanes)
    keys = jnp.where(within, keyfix, IMAX)
    cols = lax.broadcasted_iota(jnp.int32, (QB, NSAMPLE), 1)

    def pick_step(k, carry):
        keys, sel = carry
        mk = jnp.min(keys, axis=1, keepdims=True)
        p = jnp.bitwise_and(mk, np.int32(0xFFF))
        sel = jnp.where(cols == k, p + jnp.zeros((QB, NSAMPLE), jnp.int32), sel)
        keys = jnp.where(keys == mk, IMAX, keys)
        return keys, sel

    sel0 = jnp.zeros((QB, NSAMPLE), dtype=jnp.int32)
    carry1 = pick_step(0, (keys, sel0))
    _, sel = lax.fori_loop(1, NSAMPLE, pick_step, carry1)

    mod = cols % jnp.maximum(v, 1)
    zi = jnp.zeros((QB, NSAMPLE), dtype=jnp.int32)
    res = zi
    for j in range(NSAMPLE):
        res = jnp.where(mod == j, sel[:, j:j + 1] + zi, res)
    res = jnp.where(v == 0, closest + zi, res)
    sel_ref[0, 0] = res + b * N


def _ball_query(xyz_t, new_xyz, half):
    return pl.pallas_call(
        _bq_body,
        grid=(B,),
        in_specs=[
            pl.BlockSpec((1, 3, N), lambda b: (b, 0, 0)),
            pl.BlockSpec((1, QB, 3), lambda b: (b, half, 0)),
        ],
        out_specs=pl.BlockSpec((1, 1, QB, NSAMPLE), lambda b: (b, 0, 0, 0)),
        out_shape=jax.ShapeDtypeStruct((B, 1, QB, NSAMPLE), jnp.int32),
    )(xyz_t, new_xyz)


GCHUNK = 128
GBUF = 4


def _sc_gather(p_flat, ids2d, nrows):
    info = plsc.get_sparse_core_info()
    nc, ns = info.num_cores, info.num_subcores
    nw = nc * ns
    chunks_per_w = nrows // nw // GCHUNK
    outer = chunks_per_w // GBUF
    mesh = plsc.VectorSubcoreMesh(core_axis_name="c", subcore_axis_name="s")

    @functools.partial(
        pl.kernel,
        mesh=mesh,
        out_type=jax.ShapeDtypeStruct((nrows, 128), jnp.float32),
        scratch_types=[
            pltpu.VMEM((chunks_per_w, GCHUNK), jnp.int32),
            pltpu.VMEM((GBUF, GCHUNK, 128), jnp.float32),
            pltpu.SemaphoreType.DMA,
            pltpu.SemaphoreType.DMA,
        ],
    )
    def k(p_hbm, idx_hbm, out_hbm, idx_v, rows_v, gsem, wsem):
        wid = lax.axis_index("s") * nc + lax.axis_index("c")
        pltpu.sync_copy(idx_hbm.at[pl.ds(wid * chunks_per_w, chunks_per_w)], idx_v)

        def body(g, _):
            copies = []
            for bb in range(GBUF):
                copies.append(pltpu.async_copy(
                    p_hbm.at[idx_v.at[g * GBUF + bb]], rows_v.at[bb], gsem))
            wcopies = []
            for bb in range(GBUF):
                copies[bb].wait()
                base = (wid * chunks_per_w + g * GBUF + bb) * GCHUNK
                wcopies.append(pltpu.async_copy(
                    rows_v.at[bb], out_hbm.at[pl.ds(base, GCHUNK)], wsem))
            for bb in range(GBUF):
                wcopies[bb].wait()
            return 0

        lax.fori_loop(0, outer, body, 0)

    return k(p_flat, ids2d)


MBLK = 64


def _stats0_body(g_ref, c_ref, s_ref):
    y = g_ref[..., :64] - c_ref[..., :64][:, None, :]
    s = jnp.sum(y, axis=(0, 1))
    sq = jnp.sum(y * y, axis=(0, 1))
    acc = jnp.stack([s, sq])

    @pl.when(pl.program_id(0) == 0)
    def _():
        s_ref[...] = jnp.zeros_like(s_ref)

    s_ref[...] += acc


def _stats0(g, cq, nq):
    return pl.pallas_call(
        _stats0_body,
        grid=(nq // MBLK,),
        in_specs=[
            pl.BlockSpec((MBLK, NSAMPLE, 128), lambda i: (i, 0, 0)),
            pl.BlockSpec((MBLK, 128), lambda i: (i, 0)),
        ],
        out_specs=pl.BlockSpec((2, 64), lambda i: (0, 0)),
        out_shape=jax.ShapeDtypeStruct((2, 64), jnp.float32),
    )(g, cq)


def _norm_consts(sums, gamma, beta):
    cnt = np.float32(NROWS)
    mean = sums[0] / cnt
    var = sums[1] / cnt - mean * mean
    scale = gamma / jnp.sqrt(var + EPS)
    shift = beta - mean * scale
    return scale, shift


def _layer_body(sub_cq, cin, g_ref, c_ref, s_ref, w_ref, gm_ref, bt_ref, o_ref, acc_ref):
    scale, shift = _norm_consts(s_ref[...], gm_ref[0], bt_ref[0])
    x = g_ref[..., :cin]
    if sub_cq:
        x = x - c_ref[..., :cin][:, None, :]
    h = jnp.maximum(x * scale + shift, 0.0)
    h2 = h.reshape(MBLK * NSAMPLE, cin)
    y = lax.dot_general(h2, w_ref[...], (((1,), (1,)), ((), ())),
                        preferred_element_type=jnp.float32)
    cout = y.shape[-1]
    o_ref[...] = y.reshape(MBLK, NSAMPLE, cout)
    s = jnp.sum(y, axis=0)
    sq = jnp.sum(y * y, axis=0)
    acc = jnp.stack([s, sq])

    @pl.when(pl.program_id(0) == 0)
    def _():
        acc_ref[...] = jnp.zeros_like(acc_ref)

    acc_ref[...] += acc


def _layer(g, cq, sums, w, gamma, beta, sub_cq, cin, nq):
    cout = w.shape[0]
    gw = g.shape[-1]
    cw = cq.shape[-1]
    in_specs = [
        pl.BlockSpec((MBLK, NSAMPLE, gw), lambda i: (i, 0, 0)),
        pl.BlockSpec((MBLK, cw), lambda i: (i, 0)),
        pl.BlockSpec((2, cin), lambda i: (0, 0)),
        pl.BlockSpec((cout, cin), lambda i: (0, 0)),
        pl.BlockSpec((1, cin), lambda i: (0, 0)),
        pl.BlockSpec((1, cin), lambda i: (0, 0)),
    ]
    return pl.pallas_call(
        functools.partial(_layer_body, sub_cq, cin),
        grid=(nq // MBLK,),
        in_specs=in_specs,
        out_specs=(
            pl.BlockSpec((MBLK, NSAMPLE, cout), lambda i: (i, 0, 0)),
            pl.BlockSpec((2, cout), lambda i: (0, 0)),
        ),
        out_shape=(
            jax.ShapeDtypeStruct((nq, NSAMPLE, cout), jnp.float32),
            jax.ShapeDtypeStruct((2, cout), jnp.float32),
        ),
    )(g, cq, sums, w, gamma.reshape(1, cin), beta.reshape(1, cin))


def _final_body(g_ref, s_ref, gm_ref, bt_ref, o_ref):
    scale, shift = _norm_consts(s_ref[...], gm_ref[0], bt_ref[0])
    h = jnp.maximum(g_ref[...] * scale + shift, 0.0)
    o_ref[...] = jnp.max(h, axis=1)


def _final(y2, sums, gamma, beta, nq):
    c = y2.shape[-1]
    return pl.pallas_call(
        _final_body,
        grid=(nq // MBLK,),
        in_specs=[
            pl.BlockSpec((MBLK, NSAMPLE, c), lambda i: (i, 0, 0)),
            pl.BlockSpec((2, c), lambda i: (0, 0)),
            pl.BlockSpec((1, c), lambda i: (0, 0)),
            pl.BlockSpec((1, c), lambda i: (0, 0)),
        ],
        out_specs=pl.BlockSpec((MBLK, c), lambda i: (i, 0)),
        out_shape=jax.ShapeDtypeStruct((nq, c), jnp.float32),
    )(y2, sums, gamma.reshape(1, c), beta.reshape(1, c))


def kernel(xyz, points, W0, b0, g0, beta0, W1, b1, g1, beta1, W2, b2, g2, beta2):
    del b0, b1, b2
    xyz_t3 = jnp.transpose(xyz, (2, 0, 1))
    new_xyz_t, _ = _fps(xyz_t3)
    new_xyz = jnp.transpose(new_xyz_t, (0, 2, 1))

    a_pts = jnp.concatenate([xyz, points], axis=-1).reshape(B * N, 3 + points.shape[-1])
    a_pts = jnp.pad(a_pts, ((0, 0), (0, 128 - a_pts.shape[1])))
    a_ctr = jnp.pad(new_xyz.reshape(NQ, 3), ((0, 0), (0, 125)))
    a_all = jnp.concatenate([a_pts, a_ctr], axis=0)
    w0_pad = jnp.pad(W0, ((0, 128 - W0.shape[0]), (0, 128 - W0.shape[1])))
    p_all = _proj(a_all, w0_pad)
    p_flat = p_all[:B * N]
    cq = p_all[B * N:]

    xyz_t = jnp.transpose(xyz, (0, 2, 1))
    nh = NQ // 2
    nrh = NROWS // 2
    cq2 = cq.reshape(B, NPOINT, 128)
    gs, cqs = [], []
    for h in range(2):
        selg = _ball_query(xyz_t, new_xyz, h)
        gs.append(_sc_gather(
            p_flat, selg.reshape(nrh // GCHUNK, GCHUNK), nrh).reshape(
            nh, NSAMPLE, 128))
        cqs.append(cq2[:, h * QB:(h + 1) * QB].reshape(nh, 128))

    sums0 = _stats0(gs[0], cqs[0], nh) + _stats0(gs[1], cqs[1], nh)
    y1s, s1 = [], []
    for h in range(2):
        y1, p1 = _layer(gs[h], cqs[h], sums0, W1, g0, beta0, True, 64, nh)
        y1s.append(y1)
        s1.append(p1)
    sums1 = s1[0] + s1[1]
    y2s, s2 = [], []
    for h in range(2):
        y2, p2 = _layer(y1s[h], cqs[h], sums1, W2, g1, beta1, False, 64, nh)
        y2s.append(y2)
        s2.append(p2)
    sums2 = s2[0] + s2[1]
    outs = [_final(y2s[h], sums2, g2, beta2, nh).reshape(B, QB, -1)
            for h in range(2)]
    new_points = jnp.concatenate(outs, axis=1)
    return new_xyz, new_points


# speedup vs baseline: 1.0138x; 1.0006x over previous
import functools

import jax
import jax.numpy as jnp
import numpy as np
from jax import lax
from jax.experimental import pallas as pl
from jax.experimental.pallas import tpu as pltpu
from jax.experimental.pallas import tpu_sc as plsc

B = 8
N = 4096
NPOINT = 512
NSAMPLE = 32
R2 = np.float32(0.2 * 0.2)
EPS = np.float32(1e-5)
NROWS = B * NPOINT * NSAMPLE
NQ = B * NPOINT
BIG = np.int32(1 << 30)


def _fps_body(xyz_ref, nxt_ref, idx_ref):
    X = xyz_ref[0]
    Y = xyz_ref[1]
    Z = xyz_ref[2]
    lanes = lax.broadcasted_iota(jnp.int32, (B, N), 1)
    slots = lax.broadcasted_iota(jnp.int32, (B, NPOINT), 1)

    def step(i, carry):
        dist, far, cxs, cys, czs, idxs = carry
        oh = lanes == far
        cx = jnp.sum(jnp.where(oh, X, 0.0), axis=1, keepdims=True)
        cy = jnp.sum(jnp.where(oh, Y, 0.0), axis=1, keepdims=True)
        cz = jnp.sum(jnp.where(oh, Z, 0.0), axis=1, keepdims=True)
        hit = slots == i
        zf = jnp.zeros((B, NPOINT), jnp.float32)
        cxs = jnp.where(hit, cx + zf, cxs)
        cys = jnp.where(hit, cy + zf, cys)
        czs = jnp.where(hit, cz + zf, czs)
        idxs = jnp.where(hit, far + jnp.zeros((B, NPOINT), jnp.int32), idxs)
        dx = X - cx
        dy = Y - cy
        dz = Z - cz
        d = (dx * dx + dy * dy) + dz * dz
        dist = jnp.minimum(dist, d)
        m = jnp.max(dist, axis=1, keepdims=True)
        far = jnp.min(jnp.where(dist == m, lanes, BIG), axis=1, keepdims=True)
        return dist, far, cxs, cys, czs, idxs

    d0 = jnp.full((B, N), 1e10, dtype=jnp.float32)
    f0 = jnp.zeros((B, 1), dtype=jnp.int32)
    z0 = jnp.zeros((B, NPOINT), dtype=jnp.float32)
    i0 = jnp.zeros((B, NPOINT), dtype=jnp.int32)
    carry1 = step(0, (d0, f0, z0, z0, z0, i0))
    _, _, cxs, cys, czs, idxs = lax.fori_loop(1, NPOINT, step, carry1)
    nxt_ref[:, 0, :] = cxs
    nxt_ref[:, 1, :] = cys
    nxt_ref[:, 2, :] = czs
    idx_ref[...] = idxs


def _fps(xyz_t):
    return pl.pallas_call(
        _fps_body,
        out_shape=(
            jax.ShapeDtypeStruct((B, 3, NPOINT), jnp.float32),
            jax.ShapeDtypeStruct((B, NPOINT), jnp.int32),
        ),
    )(xyz_t)


def _proj_body(a_ref, w_ref, o_ref):
    o_ref[...] = lax.dot_general(
        a_ref[...], w_ref[...], (((1,), (1,)), ((), ())),
        preferred_element_type=jnp.float32)


def _proj(a_pad, w_pad):
    rtot = a_pad.shape[0]
    blk = 1024
    return pl.pallas_call(
        _proj_body,
        grid=(rtot // blk,),
        in_specs=[
            pl.BlockSpec((blk, 128), lambda i: (i, 0)),
            pl.BlockSpec((128, 128), lambda i: (0, 0)),
        ],
        out_specs=pl.BlockSpec((blk, 128), lambda i: (i, 0)),
        out_shape=jax.ShapeDtypeStruct((rtot, 128), jnp.float32),
    )(a_pad, w_pad)


QB = 256
IMAX = np.int32(0x7FFFFFFF)
KFIX = np.float32((1 << 19) / (0.2 * 0.2))


def _bq_body(xyz_ref, q_ref, sel_ref):
    b = pl.program_id(0)
    X = xyz_ref[0, 0:1, :]
    Y = xyz_ref[0, 1:2, :]
    Z = xyz_ref[0, 2:3, :]
    qx = q_ref[0, :, 0:1]
    qy = q_ref[0, :, 1:2]
    qz = q_ref[0, :, 2:3]
    dx = qx - X
    dy = qy - Y
    dz = qz - Z
    d = (dx * dx + dy * dy) + dz * dz
    lanes = lax.broadcasted_iota(jnp.int32, (QB, N), 1)
    db = lax.bitcast_convert_type(d, jnp.int32)
    keyfull = jnp.bitwise_or(jnp.bitwise_and(db, np.int32(~0xFFF)), lanes)
    closest = jnp.bitwise_and(
        jnp.min(keyfull, axis=1, keepdims=True), np.int32(0xFFF))
    within = d < R2
    v = jnp.sum(within.astype(jnp.int32), axis=1, keepdims=True)
    df = jnp.minimum((d * KFIX).astype(jnp.int32), np.int32((1 << 19) - 1))
    keyfix = jnp.bitwise_or(lax.shift_left(df, 12), lanes)
    keys = jnp.where(within, keyfix, IMAX)
    cols = lax.broadcasted_iota(jnp.int32, (QB, NSAMPLE), 1)

    def pick_step(k, carry):
        keys, sel = carry
        mk = jnp.min(keys, axis=1, keepdims=True)
        p = jnp.bitwise_and(mk, np.int32(0xFFF))
        sel = jnp.where(cols == k, p + jnp.zeros((QB, NSAMPLE), jnp.int32), sel)
        keys = jnp.where(keys == mk, IMAX, keys)
        return keys, sel

    sel0 = jnp.zeros((QB, NSAMPLE), dtype=jnp.int32)
    carry1 = pick_step(0, (keys, sel0))
    _, sel = lax.fori_loop(1, NSAMPLE, pick_step, carry1)

    mod = cols % jnp.maximum(v, 1)
    zi = jnp.zeros((QB, NSAMPLE), dtype=jnp.int32)
    res = zi
    for j in range(NSAMPLE):
        res = jnp.where(mod == j, sel[:, j:j + 1] + zi, res)
    res = jnp.where(v == 0, closest + zi, res)
    sel_ref[0, 0] = res + b * N


def _ball_query(xyz_t, new_xyz, half):
    return pl.pallas_call(
        _bq_body,
        grid=(B,),
        in_specs=[
            pl.BlockSpec((1, 3, N), lambda b: (b, 0, 0)),
            pl.BlockSpec((1, QB, 3), lambda b: (b, half, 0)),
        ],
        out_specs=pl.BlockSpec((1, 1, QB, NSAMPLE), lambda b: (b, 0, 0, 0)),
        out_shape=jax.ShapeDtypeStruct((B, 1, QB, NSAMPLE), jnp.int32),
    )(xyz_t, new_xyz)


GCHUNK = 128
GBUF = 4


def _sc_gather(p_flat, ids2d, nrows):
    info = plsc.get_sparse_core_info()
    nc, ns = info.num_cores, info.num_subcores
    nw = nc * ns
    chunks_per_w = nrows // nw // GCHUNK
    outer = chunks_per_w // GBUF
    mesh = plsc.VectorSubcoreMesh(core_axis_name="c", subcore_axis_name="s")

    @functools.partial(
        pl.kernel,
        mesh=mesh,
        out_type=jax.ShapeDtypeStruct((nrows, 128), jnp.float32),
        scratch_types=[
            pltpu.VMEM((chunks_per_w, GCHUNK), jnp.int32),
            pltpu.VMEM((GBUF, GCHUNK, 128), jnp.float32),
            pltpu.SemaphoreType.DMA,
            pltpu.SemaphoreType.DMA,
            pltpu.SemaphoreType.DMA,
            pltpu.SemaphoreType.DMA,
        ],
    )
    def k(p_hbm, idx_hbm, out_hbm, idx_v, rows_v, s0, s1, s2, s3):
        sems = (s0, s1, s2, s3)
        wid = lax.axis_index("s") * nc + lax.axis_index("c")
        pltpu.sync_copy(idx_hbm.at[pl.ds(wid * chunks_per_w, chunks_per_w)], idx_v)

        def body(g, _):
            copies = []
            for bb in range(GBUF):
                copies.append(pltpu.async_copy(
                    p_hbm.at[idx_v.at[g * GBUF + bb]], rows_v.at[bb], sems[bb]))
            for bb in range(GBUF):
                copies[bb].wait()
                base = (wid * chunks_per_w + g * GBUF + bb) * GCHUNK
                pltpu.sync_copy(rows_v.at[bb], out_hbm.at[pl.ds(base, GCHUNK)])
            return 0

        lax.fori_loop(0, outer, body, 0)

    return k(p_flat, ids2d)


MBLK = 64


def _stats0_body(g_ref, c_ref, s_ref):
    y = g_ref[..., :64] - c_ref[..., :64][:, None, :]
    s = jnp.sum(y, axis=(0, 1))
    sq = jnp.sum(y * y, axis=(0, 1))
    acc = jnp.stack([s, sq])

    @pl.when(pl.program_id(0) == 0)
    def _():
        s_ref[...] = jnp.zeros_like(s_ref)

    s_ref[...] += acc


def _stats0(g, cq, nq):
    return pl.pallas_call(
        _stats0_body,
        grid=(nq // MBLK,),
        in_specs=[
            pl.BlockSpec((MBLK, NSAMPLE, 128), lambda i: (i, 0, 0)),
            pl.BlockSpec((MBLK, 128), lambda i: (i, 0)),
        ],
        out_specs=pl.BlockSpec((2, 64), lambda i: (0, 0)),
        out_shape=jax.ShapeDtypeStruct((2, 64), jnp.float32),
    )(g, cq)


def _norm_consts(sums, gamma, beta):
    cnt = np.float32(NROWS)
    mean = sums[0] / cnt
    var = sums[1] / cnt - mean * mean
    scale = gamma / jnp.sqrt(var + EPS)
    shift = beta - mean * scale
    return scale, shift


def _layer_body(sub_cq, cin, g_ref, c_ref, s_ref, w_ref, gm_ref, bt_ref, o_ref, acc_ref):
    scale, shift = _norm_consts(s_ref[...], gm_ref[0], bt_ref[0])
    x = g_ref[..., :cin]
    if sub_cq:
        x = x - c_ref[..., :cin][:, None, :]
    h = jnp.maximum(x * scale + shift, 0.0)
    h2 = h.reshape(MBLK * NSAMPLE, cin)
    y = lax.dot_general(h2, w_ref[...], (((1,), (1,)), ((), ())),
                        preferred_element_type=jnp.float32)
    cout = y.shape[-1]
    o_ref[...] = y.reshape(MBLK, NSAMPLE, cout)
    s = jnp.sum(y, axis=0)
    sq = jnp.sum(y * y, axis=0)
    acc = jnp.stack([s, sq])

    @pl.when(pl.program_id(0) == 0)
    def _():
        acc_ref[...] = jnp.zeros_like(acc_ref)

    acc_ref[...] += acc


def _layer(g, cq, sums, w, gamma, beta, sub_cq, cin, nq):
    cout = w.shape[0]
    gw = g.shape[-1]
    cw = cq.shape[-1]
    in_specs = [
        pl.BlockSpec((MBLK, NSAMPLE, gw), lambda i: (i, 0, 0)),
        pl.BlockSpec((MBLK, cw), lambda i: (i, 0)),
        pl.BlockSpec((2, cin), lambda i: (0, 0)),
        pl.BlockSpec((cout, cin), lambda i: (0, 0)),
        pl.BlockSpec((1, cin), lambda i: (0, 0)),
        pl.BlockSpec((1, cin), lambda i: (0, 0)),
    ]
    return pl.pallas_call(
        functools.partial(_layer_body, sub_cq, cin),
        grid=(nq // MBLK,),
        in_specs=in_specs,
        out_specs=(
            pl.BlockSpec((MBLK, NSAMPLE, cout), lambda i: (i, 0, 0)),
            pl.BlockSpec((2, cout), lambda i: (0, 0)),
        ),
        out_shape=(
            jax.ShapeDtypeStruct((nq, NSAMPLE, cout), jnp.float32),
            jax.ShapeDtypeStruct((2, cout), jnp.float32),
        ),
    )(g, cq, sums, w, gamma.reshape(1, cin), beta.reshape(1, cin))


def _final_body(g_ref, s_ref, gm_ref, bt_ref, o_ref):
    scale, shift = _norm_consts(s_ref[...], gm_ref[0], bt_ref[0])
    h = jnp.maximum(g_ref[...] * scale + shift, 0.0)
    o_ref[...] = jnp.max(h, axis=1)


def _final(y2, sums, gamma, beta, nq):
    c = y2.shape[-1]
    return pl.pallas_call(
        _final_body,
        grid=(nq // MBLK,),
        in_specs=[
            pl.BlockSpec((MBLK, NSAMPLE, c), lambda i: (i, 0, 0)),
            pl.BlockSpec((2, c), lambda i: (0, 0)),
            pl.BlockSpec((1, c), lambda i: (0, 0)),
            pl.BlockSpec((1, c), lambda i: (0, 0)),
        ],
        out_specs=pl.BlockSpec((MBLK, c), lambda i: (i, 0)),
        out_shape=jax.ShapeDtypeStruct((nq, c), jnp.float32),
    )(y2, sums, gamma.reshape(1, c), beta.reshape(1, c))


def kernel(xyz, points, W0, b0, g0, beta0, W1, b1, g1, beta1, W2, b2, g2, beta2):
    del b0, b1, b2
    xyz_t3 = jnp.transpose(xyz, (2, 0, 1))
    new_xyz_t, _ = _fps(xyz_t3)
    new_xyz = jnp.transpose(new_xyz_t, (0, 2, 1))

    a_pts = jnp.concatenate([xyz, points], axis=-1).reshape(B * N, 3 + points.shape[-1])
    a_pts = jnp.pad(a_pts, ((0, 0), (0, 128 - a_pts.shape[1])))
    a_ctr = jnp.pad(new_xyz.reshape(NQ, 3), ((0, 0), (0, 125)))
    a_all = jnp.concatenate([a_pts, a_ctr], axis=0)
    w0_pad = jnp.pad(W0, ((0, 128 - W0.shape[0]), (0, 128 - W0.shape[1])))
    p_all = _proj(a_all, w0_pad)
    p_flat = p_all[:B * N]
    cq = p_all[B * N:]

    xyz_t = jnp.transpose(xyz, (0, 2, 1))
    nh = NQ // 2
    nrh = NROWS // 2
    cq2 = cq.reshape(B, NPOINT, 128)
    gs, cqs = [], []
    for h in range(2):
        selg = _ball_query(xyz_t, new_xyz, h)
        gs.append(_sc_gather(
            p_flat, selg.reshape(nrh // GCHUNK, GCHUNK), nrh).reshape(
            nh, NSAMPLE, 128))
        cqs.append(cq2[:, h * QB:(h + 1) * QB].reshape(nh, 128))

    sums0 = _stats0(gs[0], cqs[0], nh) + _stats0(gs[1], cqs[1], nh)
    y1s, s1 = [], []
    for h in range(2):
        y1, p1 = _layer(gs[h], cqs[h], sums0, W1, g0, beta0, True, 64, nh)
        y1s.append(y1)
        s1.append(p1)
    sums1 = s1[0] + s1[1]
    y2s, s2 = [], []
    for h in range(2):
        y2, p2 = _layer(y1s[h], cqs[h], sums1, W2, g1, beta1, False, 64, nh)
        y2s.append(y2)
        s2.append(p2)
    sums2 = s2[0] + s2[1]
    outs = [_final(y2s[h], sums2, g2, beta2, nh).reshape(B, QB, -1)
            for h in range(2)]
    new_points = jnp.concatenate(outs, axis=1)
    return new_xyz, new_points
